# baseline (speedup 1.0000x reference)
import functools

import jax
import jax.numpy as jnp
from jax import lax
from jax.experimental import pallas as pl
from jax.experimental.pallas import tpu as pltpu
from jax.experimental.pallas import tpu_sc as plsc

_NW = 32
_CH = 128
_K = 8
_BLK = _CH * _K


def _sc_gather(table, idx3, n_rows, d):
    nw, nb, blk = idx3.shape
    mesh = plsc.VectorSubcoreMesh(core_axis_name="c", subcore_axis_name="s")

    @functools.partial(
        pl.kernel,
        mesh=mesh,
        out_type=jax.ShapeDtypeStruct((n_rows, d), table.dtype),
        scratch_types=[
            pltpu.VMEM((nb, blk), jnp.int32),
            pltpu.VMEM((blk, d), table.dtype),
            pltpu.SemaphoreType.DMA,
        ],
        compiler_params=pltpu.CompilerParams(use_tc_tiling_on_sc=False),
    )
    def k(table_hbm, idx_hbm, out_hbm, idx_v, rows_v, sem):
        wid = lax.axis_index("s") * 2 + lax.axis_index("c")
        pltpu.sync_copy(idx_hbm.at[wid], idx_v)
        base = wid * (nb * blk)

        @pl.loop(0, nb)
        def _(j):
            copies = []
            for u in range(_K):
                copies.append(pltpu.async_copy(
                    table_hbm.at[idx_v.at[j, pl.ds(u * _CH, _CH)]],
                    rows_v.at[pl.ds(u * _CH, _CH)],
                    sem,
                ))
            for c in copies:
                c.wait()
            pltpu.sync_copy(rows_v, out_hbm.at[pl.ds(base + j * blk, blk)])

    return k(table, idx3)


def _tc_mlp(x_deep, xn_p, W1, b1, W2, b2, W3a, W3b_p, b3, bm):
    b_total, fd = x_deep.shape
    h1 = W1.shape[1]
    h2 = W2.shape[1]
    out = W3a.shape[1]
    npad = xn_p.shape[1]

    def body(x_ref, xn_ref, w1_ref, b1_ref, w2_ref, b2_ref, w3a_ref,
             w3b_ref, b3_ref, o_ref):
        h = jnp.dot(x_ref[...], w1_ref[...],
                    preferred_element_type=jnp.float32)
        h = jnp.maximum(h + b1_ref[...], 0.0)
        h = jnp.dot(h, w2_ref[...], preferred_element_type=jnp.float32)
        h = jnp.maximum(h + b2_ref[...], 0.0)
        o = jnp.dot(h, w3a_ref[...], preferred_element_type=jnp.float32)
        o = o + jnp.dot(xn_ref[...], w3b_ref[...],
                        preferred_element_type=jnp.float32)
        o_ref[...] = o + b3_ref[...]

    return pl.pallas_call(
        body,
        grid=(b_total // bm,),
        in_specs=[
            pl.BlockSpec((bm, fd), lambda i: (i, 0)),
            pl.BlockSpec((bm, npad), lambda i: (i, 0)),
            pl.BlockSpec((fd, h1), lambda i: (0, 0)),
            pl.BlockSpec((1, h1), lambda i: (0, 0)),
            pl.BlockSpec((h1, h2), lambda i: (0, 0)),
            pl.BlockSpec((1, h2), lambda i: (0, 0)),
            pl.BlockSpec((h2, out), lambda i: (0, 0)),
            pl.BlockSpec((npad, out), lambda i: (0, 0)),
            pl.BlockSpec((1, out), lambda i: (0, 0)),
        ],
        out_specs=pl.BlockSpec((bm, out), lambda i: (i, 0)),
        out_shape=jax.ShapeDtypeStruct((b_total, out), jnp.float32),
    )(x_deep, xn_p, W1, b1, W2, b2, W3a, W3b_p, b3)


def kernel(x_cat, x_num, emb, W1, b1, W2, b2, W3, b3):
    b, f = x_cat.shape
    v, d = emb.shape[1], emb.shape[2]
    h2 = W2.shape[1]
    num = x_num.shape[1]

    one = 1.0 + 0.0 * W1[0, 0]
    table = (emb * one).reshape(f * v, d)
    idx = x_cat.astype(jnp.int32) + (jnp.arange(f, dtype=jnp.int32) * v)[None, :]
    n_rows = b * f
    idx3 = idx.reshape(_NW, n_rows // (_NW * _BLK), _BLK)

    gathered = _sc_gather(table, idx3, n_rows, d)
    x_deep = gathered.reshape(b, f * d)

    npad = 16
    xn_p = jnp.pad(x_num, ((0, 0), (0, npad - num)))
    W3a = W3[:h2]
    W3b_p = jnp.pad(W3[h2:], ((0, npad - num), (0, 0)))

    return _tc_mlp(x_deep, xn_p, W1, b1.reshape(1, -1), W2, b2.reshape(1, -1),
                   W3a, W3b_p, b3.reshape(1, -1), bm=1024)


# speedup vs baseline: 1.1546x; 1.1546x over previous
import functools

import jax
import jax.numpy as jnp
from jax import lax
from jax.experimental import pallas as pl
from jax.experimental.pallas import tpu as pltpu
from jax.experimental.pallas import tpu_sc as plsc

_NW = 32
_CH = 128
_K = 8
_BLK = _CH * _K


def _sc_gather(table, idx3, n_rows, d):
    nw, nb, blk = idx3.shape
    mesh = plsc.VectorSubcoreMesh(core_axis_name="c", subcore_axis_name="s")

    @functools.partial(
        pl.kernel,
        mesh=mesh,
        out_type=jax.ShapeDtypeStruct((n_rows, d), table.dtype),
        scratch_types=[
            pltpu.VMEM((nb, blk), jnp.int32),
            pltpu.VMEM((blk, d), table.dtype),
            pltpu.SemaphoreType.DMA,
        ],
        compiler_params=pltpu.CompilerParams(use_tc_tiling_on_sc=False),
    )
    def k(table_hbm, idx_hbm, out_hbm, idx_v, rows_v, sem):
        wid = lax.axis_index("s") * 2 + lax.axis_index("c")
        pltpu.sync_copy(idx_hbm.at[wid], idx_v)
        base = wid * (nb * blk)

        @pl.loop(0, nb)
        def _(j):
            copies = []
            for u in range(_K):
                copies.append(pltpu.async_copy(
                    table_hbm.at[idx_v.at[j, pl.ds(u * _CH, _CH)]],
                    rows_v.at[pl.ds(u * _CH, _CH)],
                    sem,
                ))
            for c in copies:
                c.wait()
            pltpu.sync_copy(rows_v, out_hbm.at[pl.ds(base + j * blk, blk)])

    return k(table, idx3)


def _tc_mlp(x_deep, xn_p, W1, b1, W2, b2, W3a, W3b_p, b3, bm):
    b_total, fd = x_deep.shape
    h1 = W1.shape[1]
    h2 = W2.shape[1]
    out = W3a.shape[1]
    npad = xn_p.shape[1]

    def body(x_ref, xn_ref, w1_ref, b1_ref, w2_ref, b2_ref, w3a_ref,
             w3b_ref, b3_ref, o_ref):
        h = jnp.dot(x_ref[...], w1_ref[...],
                    preferred_element_type=jnp.float32)
        h = jnp.maximum(h + b1_ref[...], 0.0)
        h = jnp.dot(h, w2_ref[...], preferred_element_type=jnp.float32)
        h = jnp.maximum(h + b2_ref[...], 0.0)
        o = jnp.dot(h, w3a_ref[...], preferred_element_type=jnp.float32)
        o = o + jnp.dot(xn_ref[...], w3b_ref[...],
                        preferred_element_type=jnp.float32)
        o_ref[...] = o + b3_ref[...]

    return pl.pallas_call(
        body,
        grid=(b_total // bm,),
        in_specs=[
            pl.BlockSpec((bm, fd), lambda i: (i, 0)),
            pl.BlockSpec((bm, npad), lambda i: (i, 0)),
            pl.BlockSpec((fd, h1), lambda i: (0, 0)),
            pl.BlockSpec((1, h1), lambda i: (0, 0)),
            pl.BlockSpec((h1, h2), lambda i: (0, 0)),
            pl.BlockSpec((1, h2), lambda i: (0, 0)),
            pl.BlockSpec((h2, out), lambda i: (0, 0)),
            pl.BlockSpec((npad, out), lambda i: (0, 0)),
            pl.BlockSpec((1, out), lambda i: (0, 0)),
        ],
        out_specs=pl.BlockSpec((bm, out), lambda i: (i, 0)),
        out_shape=jax.ShapeDtypeStruct((b_total, out), jnp.float32),
    )(x_deep, xn_p, W1, b1, W2, b2, W3a, W3b_p, b3)


def kernel(x_cat, x_num, emb, W1, b1, W2, b2, W3, b3):
    b, f = x_cat.shape
    v, d = emb.shape[1], emb.shape[2]
    h2 = W2.shape[1]
    num = x_num.shape[1]

    table = emb.reshape(f * v, d)
    idx = x_cat.astype(jnp.int32) + (jnp.arange(f, dtype=jnp.int32) * v)[None, :]
    n_rows = b * f
    idx3 = idx.reshape(_NW, n_rows // (_NW * _BLK), _BLK)

    gathered = _sc_gather(table, idx3, n_rows, d)
    x_deep = gathered.reshape(b, f * d)

    npad = 16
    xn_p = jnp.pad(x_num, ((0, 0), (0, npad - num)))
    W3a = W3[:h2]
    W3b_p = jnp.pad(W3[h2:], ((0, npad - num), (0, 0)))

    return _tc_mlp(x_deep, xn_p, W1, b1.reshape(1, -1), W2, b2.reshape(1, -1),
                   W3a, W3b_p, b3.reshape(1, -1), bm=1024)


# speedup vs baseline: 1.1567x; 1.0018x over previous
import functools

import jax
import jax.numpy as jnp
from jax import lax
from jax.experimental import pallas as pl
from jax.experimental.pallas import tpu as pltpu
from jax.experimental.pallas import tpu_sc as plsc

_NW = 32
_CH = 128
_K = 13
_BLK = _CH * _K


def _sc_gather(table, idx3, n_rows, d):
    nw, nb, blk = idx3.shape
    mesh = plsc.VectorSubcoreMesh(core_axis_name="c", subcore_axis_name="s")

    @functools.partial(
        pl.kernel,
        mesh=mesh,
        out_type=jax.ShapeDtypeStruct((n_rows, d), table.dtype),
        scratch_types=[
            pltpu.VMEM((nb, blk), jnp.int32),
            pltpu.VMEM((blk, d), table.dtype),
            pltpu.SemaphoreType.DMA,
        ],
        compiler_params=pltpu.CompilerParams(use_tc_tiling_on_sc=False),
    )
    def k(table_hbm, idx_hbm, out_hbm, idx_v, rows_v, sem):
        wid = lax.axis_index("s") * 2 + lax.axis_index("c")
        pltpu.sync_copy(idx_hbm.at[wid], idx_v)
        base = wid * (nb * blk)

        @pl.loop(0, nb)
        def _(j):
            copies = []
            for u in range(_K):
                copies.append(pltpu.async_copy(
                    table_hbm.at[idx_v.at[j, pl.ds(u * _CH, _CH)]],
                    rows_v.at[pl.ds(u * _CH, _CH)],
                    sem,
                ))
            for c in copies:
                c.wait()
            pltpu.sync_copy(rows_v, out_hbm.at[pl.ds(base + j * blk, blk)])

    return k(table, idx3)


def _tc_mlp(x_deep, xn_p, W1, b1, W2, b2, W3a, W3b_p, b3, bm):
    b_total, fd = x_deep.shape
    h1 = W1.shape[1]
    h2 = W2.shape[1]
    out = W3a.shape[1]
    npad = xn_p.shape[1]

    def body(x_ref, xn_ref, w1_ref, b1_ref, w2_ref, b2_ref, w3a_ref,
             w3b_ref, b3_ref, o_ref):
        h = jnp.dot(x_ref[...], w1_ref[...],
                    preferred_element_type=jnp.float32)
        h = jnp.maximum(h + b1_ref[...], 0.0)
        h = jnp.dot(h, w2_ref[...], preferred_element_type=jnp.float32)
        h = jnp.maximum(h + b2_ref[...], 0.0)
        o = jnp.dot(h, w3a_ref[...], preferred_element_type=jnp.float32)
        o = o + jnp.dot(xn_ref[...], w3b_ref[...],
                        preferred_element_type=jnp.float32)
        o_ref[...] = o + b3_ref[...]

    return pl.pallas_call(
        body,
        grid=(b_total // bm,),
        in_specs=[
            pl.BlockSpec((bm, fd), lambda i: (i, 0)),
            pl.BlockSpec((bm, npad), lambda i: (i, 0)),
            pl.BlockSpec((fd, h1), lambda i: (0, 0)),
            pl.BlockSpec((1, h1), lambda i: (0, 0)),
            pl.BlockSpec((h1, h2), lambda i: (0, 0)),
            pl.BlockSpec((1, h2), lambda i: (0, 0)),
            pl.BlockSpec((h2, out), lambda i: (0, 0)),
            pl.BlockSpec((npad, out), lambda i: (0, 0)),
            pl.BlockSpec((1, out), lambda i: (0, 0)),
        ],
        out_specs=pl.BlockSpec((bm, out), lambda i: (i, 0)),
        out_shape=jax.ShapeDtypeStruct((b_total, out), jnp.float32),
    )(x_deep, xn_p, W1, b1, W2, b2, W3a, W3b_p, b3)


def kernel(x_cat, x_num, emb, W1, b1, W2, b2, W3, b3):
    b, f = x_cat.shape
    v, d = emb.shape[1], emb.shape[2]
    h2 = W2.shape[1]
    num = x_num.shape[1]

    table = emb.reshape(f * v, d)
    idx = x_cat.astype(jnp.int32) + (jnp.arange(f, dtype=jnp.int32) * v)[None, :]
    n_rows = b * f
    idx3 = idx.reshape(_NW, n_rows // (_NW * _BLK), _BLK)

    gathered = _sc_gather(table, idx3, n_rows, d)
    x_deep = gathered.reshape(b, f * d)

    npad = 16
    xn_p = jnp.pad(x_num, ((0, 0), (0, npad - num)))
    W3a = W3[:h2]
    W3b_p = jnp.pad(W3[h2:], ((0, npad - num), (0, 0)))

    return _tc_mlp(x_deep, xn_p, W1, b1.reshape(1, -1), W2, b2.reshape(1, -1),
                   W3a, W3b_p, b3.reshape(1, -1), bm=1024)
